# baseline (device time: 79858 ns/iter reference)
import jax
import jax.numpy as jnp
from jax import lax
from jax.experimental import pallas as pl
from jax.experimental.pallas import tpu as pltpu

N_DEV = 4
SEG = 4


def kernel(t):
    m_per, n = t.shape
    m_chunk = m_per // N_DEV
    n_half = n // 2
    n_seg = n_half // SEG

    def body(t_ref, out_ref, cw_rs, ccw_rs,
             cw_s_sems, cw_r_sems, cw_ags_sems, cw_agr_sems,
             ccw_s_sems, ccw_r_sems, ccw_ags_sems, ccw_agr_sems):
        my_pos = lax.axis_index("i")
        left = jnp.mod(my_pos - 1, N_DEV)
        right = jnp.mod(my_pos + 1, N_DEV)

        def rows(c):
            return pl.ds(jnp.mod(c, N_DEV) * m_chunk, m_chunk)

        def glob_cols(dir_is_cw, s):
            base = 0 if dir_is_cw else n_half
            return pl.ds(base + s * n_seg, n_seg)

        def rs_cols(s):
            return pl.ds(s * n_seg, n_seg)

        barrier_sem = pltpu.get_barrier_semaphore()
        for nbr in [left, right]:
            pl.semaphore_signal(
                barrier_sem, inc=1,
                device_id=(nbr,), device_id_type=pl.DeviceIdType.MESH,
            )
        pl.semaphore_wait(barrier_sem, 2)

        def rs_rdma(h, dir_is_cw, s):
            buf, ssem, rsem, tgt = (
                (cw_rs, cw_s_sems, cw_r_sems, right) if dir_is_cw
                else (ccw_rs, ccw_s_sems, ccw_r_sems, left)
            )
            if h == 0:
                src = t_ref.at[rows(my_pos), glob_cols(dir_is_cw, s)]
            else:
                src = buf.at[h - 1, :, rs_cols(s)]
            return pltpu.make_async_remote_copy(
                src_ref=src, dst_ref=buf.at[h, :, rs_cols(s)],
                send_sem=ssem.at[h, s], recv_sem=rsem.at[h, s],
                device_id=(tgt,), device_id_type=pl.DeviceIdType.MESH,
            )

        def ag_rdma(h, dir_is_cw, s):
            if dir_is_cw:
                c_send, ssem, rsem, tgt = my_pos + 1 - h, cw_ags_sems, cw_agr_sems, right
            else:
                c_send, ssem, rsem, tgt = my_pos - 1 + h, ccw_ags_sems, ccw_agr_sems, left
            sl = out_ref.at[rows(c_send), glob_cols(dir_is_cw, s)]
            return pltpu.make_async_remote_copy(
                src_ref=sl, dst_ref=sl,
                send_sem=ssem.at[h, s], recv_sem=rsem.at[h, s],
                device_id=(tgt,), device_id_type=pl.DeviceIdType.MESH,
            )

        rs = {(h, d, s): rs_rdma(h, d, s)
              for h in range(N_DEV - 1) for d in (True, False) for s in range(SEG)}
        ag = {(h, d, s): ag_rdma(h, d, s)
              for h in range(N_DEV - 1) for d in (True, False) for s in range(SEG)}

        for s in range(SEG):
            rs[0, True, s].start()
            rs[0, False, s].start()

        for h in range(N_DEV - 1):
            for s in range(SEG):
                for d in (True, False):
                    rs[h, d, s].wait_recv()
                    buf = cw_rs if d else ccw_rs
                    c_in = my_pos - h - 1 if d else my_pos + h + 1
                    mine = t_ref[rows(c_in), glob_cols(d, s)]
                    if h < N_DEV - 2:
                        buf[h, :, rs_cols(s)] = buf[h, :, rs_cols(s)] + mine
                        rs[h + 1, d, s].start()
                    else:
                        sv = buf[h, :, rs_cols(s)] + mine
                        r = jnp.maximum(sv, 0.0)
                        c_own = my_pos + 1 if d else my_pos - 1
                        out_ref[rows(c_own), glob_cols(d, s)] = (
                            jnp.tanh(sv) * sv * sv + r * r * r
                        )
                        ag[0, d, s].start()

        for h in range(N_DEV - 1):
            for s in range(SEG):
                for d in (True, False):
                    ag[h, d, s].wait_recv()
                    if h < N_DEV - 2:
                        ag[h + 1, d, s].start()

        for r_ in list(rs.values()) + list(ag.values()):
            r_.wait_send()

    half = (m_chunk, n_half)
    sem = pltpu.SemaphoreType.DMA((N_DEV - 1, SEG))
    return pl.pallas_call(
        body,
        out_shape=jax.ShapeDtypeStruct((m_per, n), jnp.float32),
        in_specs=[pl.BlockSpec(memory_space=pltpu.VMEM)],
        out_specs=pl.BlockSpec(memory_space=pltpu.VMEM),
        scratch_shapes=[
            pltpu.VMEM((N_DEV - 1,) + half, jnp.float32),
            pltpu.VMEM((N_DEV - 1,) + half, jnp.float32),
        ] + [sem for _ in range(8)],
        compiler_params=pltpu.CompilerParams(collective_id=0),
    )(t)


# device time: 46258 ns/iter; 1.7264x vs baseline; 1.7264x over previous
import jax
import jax.numpy as jnp
from jax import lax
from jax.experimental import pallas as pl
from jax.experimental.pallas import tpu as pltpu

N_DEV = 4
SEG = 4


def kernel(t):
    m_per, n = t.shape
    m_chunk = m_per // N_DEV
    n_half = n // 2
    n_seg = n_half // SEG

    def body(t_ref, out_ref, cw_send, ccw_send, cw_rs, ccw_rs, cw_ag, ccw_ag,
             cw_s_sems, cw_r_sems, cw_ags_sems, cw_agr_sems,
             ccw_s_sems, ccw_r_sems, ccw_ags_sems, ccw_agr_sems):
        my_pos = lax.axis_index("i")
        left = jnp.mod(my_pos - 1, N_DEV)
        right = jnp.mod(my_pos + 1, N_DEV)

        def rows(c):
            return pl.ds(jnp.mod(c, N_DEV) * m_chunk, m_chunk)

        def glob_cols(d, s):
            return pl.ds((0 if d else n_half) + s * n_seg, n_seg)

        def seg_cols(s):
            return pl.ds(s * n_seg, n_seg)

        barrier_sem = pltpu.get_barrier_semaphore()
        for nbr in [left, right]:
            pl.semaphore_signal(
                barrier_sem, inc=1,
                device_id=(nbr,), device_id_type=pl.DeviceIdType.MESH,
            )
        pl.semaphore_wait(barrier_sem, 2)

        def rs_rdma(h, d, s):
            snd, buf, ssem, rsem, tgt = (
                (cw_send, cw_rs, cw_s_sems, cw_r_sems, right) if d
                else (ccw_send, ccw_rs, ccw_s_sems, ccw_r_sems, left)
            )
            return pltpu.make_async_remote_copy(
                src_ref=snd.at[h, :, seg_cols(s)],
                dst_ref=buf.at[h, :, seg_cols(s)],
                send_sem=ssem.at[h, s], recv_sem=rsem.at[h, s],
                device_id=(tgt,), device_id_type=pl.DeviceIdType.MESH,
            )

        def ag_rdma(h, d, s):
            buf, ssem, rsem, tgt = (
                (cw_ag, cw_ags_sems, cw_agr_sems, right) if d
                else (ccw_ag, ccw_ags_sems, ccw_agr_sems, left)
            )
            return pltpu.make_async_remote_copy(
                src_ref=buf.at[h, :, seg_cols(s)],
                dst_ref=buf.at[h + 1, :, seg_cols(s)],
                send_sem=ssem.at[h, s], recv_sem=rsem.at[h, s],
                device_id=(tgt,), device_id_type=pl.DeviceIdType.MESH,
            )

        rs = {(h, d, s): rs_rdma(h, d, s)
              for h in range(N_DEV - 1) for d in (True, False) for s in range(SEG)}
        ag = {(h, d, s): ag_rdma(h, d, s)
              for h in range(N_DEV - 1) for d in (True, False) for s in range(SEG)}

        for s in range(SEG):
            for d in (True, False):
                snd = cw_send if d else ccw_send
                snd[0, :, seg_cols(s)] = t_ref[
                    rows(my_pos), glob_cols(d, s)
                ].astype(jnp.bfloat16)
                rs[0, d, s].start()

        for h in range(N_DEV - 1):
            for s in range(SEG):
                for d in (True, False):
                    rs[h, d, s].wait_recv()
                    snd, buf = (cw_send, cw_rs) if d else (ccw_send, ccw_rs)
                    c_in = my_pos - h - 1 if d else my_pos + h + 1
                    partial = (
                        buf[h, :, seg_cols(s)].astype(jnp.float32)
                        + t_ref[rows(c_in), glob_cols(d, s)]
                    )
                    if h < N_DEV - 2:
                        snd[h + 1, :, seg_cols(s)] = partial.astype(jnp.bfloat16)
                        rs[h + 1, d, s].start()
                    else:
                        r = jnp.maximum(partial, 0.0)
                        fv = jnp.tanh(partial) * partial * partial + r * r * r
                        agb = cw_ag if d else ccw_ag
                        agb[0, :, seg_cols(s)] = fv.astype(jnp.bfloat16)
                        ag[0, d, s].start()
                        c_own = my_pos + 1 if d else my_pos - 1
                        out_ref[rows(c_own), glob_cols(d, s)] = fv

        for h in range(N_DEV - 1):
            for s in range(SEG):
                for d in (True, False):
                    ag[h, d, s].wait_recv()
                    if h < N_DEV - 2:
                        ag[h + 1, d, s].start()
                    agb = cw_ag if d else ccw_ag
                    c_recv = my_pos - h if d else my_pos + h
                    out_ref[rows(c_recv), glob_cols(d, s)] = agb[
                        h + 1, :, seg_cols(s)
                    ].astype(jnp.float32)

        for r_ in list(rs.values()) + list(ag.values()):
            r_.wait_send()

    half = (m_chunk, n_half)
    sem = pltpu.SemaphoreType.DMA((N_DEV - 1, SEG))
    return pl.pallas_call(
        body,
        out_shape=jax.ShapeDtypeStruct((m_per, n), jnp.float32),
        in_specs=[pl.BlockSpec(memory_space=pltpu.VMEM)],
        out_specs=pl.BlockSpec(memory_space=pltpu.VMEM),
        scratch_shapes=[
            pltpu.VMEM((N_DEV - 1,) + half, jnp.bfloat16),
            pltpu.VMEM((N_DEV - 1,) + half, jnp.bfloat16),
            pltpu.VMEM((N_DEV - 1,) + half, jnp.bfloat16),
            pltpu.VMEM((N_DEV - 1,) + half, jnp.bfloat16),
            pltpu.VMEM((N_DEV,) + half, jnp.bfloat16),
            pltpu.VMEM((N_DEV,) + half, jnp.bfloat16),
        ] + [sem for _ in range(8)],
        compiler_params=pltpu.CompilerParams(collective_id=0),
    )(t)


# device time: 45785 ns/iter; 1.7442x vs baseline; 1.0103x over previous
import jax
import jax.numpy as jnp
from jax import lax
from jax.experimental import pallas as pl
from jax.experimental.pallas import tpu as pltpu

N_DEV = 4
SEG = 2


def kernel(t):
    m_per, n = t.shape
    m_chunk = m_per // N_DEV
    n_half = n // 2
    n_seg = n_half // SEG

    def body(t_ref, out_ref, cw_send, ccw_send, cw_rs, ccw_rs, cw_ag, ccw_ag,
             cw_s_sems, cw_r_sems, cw_ags_sems, cw_agr_sems,
             ccw_s_sems, ccw_r_sems, ccw_ags_sems, ccw_agr_sems):
        my_pos = lax.axis_index("i")
        left = jnp.mod(my_pos - 1, N_DEV)
        right = jnp.mod(my_pos + 1, N_DEV)

        def rows(c):
            return pl.ds(jnp.mod(c, N_DEV) * m_chunk, m_chunk)

        def glob_cols(d, s):
            return pl.ds((0 if d else n_half) + s * n_seg, n_seg)

        def seg_cols(s):
            return pl.ds(s * n_seg, n_seg)

        barrier_sem = pltpu.get_barrier_semaphore()
        for nbr in [left, right]:
            pl.semaphore_signal(
                barrier_sem, inc=1,
                device_id=(nbr,), device_id_type=pl.DeviceIdType.MESH,
            )
        pl.semaphore_wait(barrier_sem, 2)

        def rs_rdma(h, d, s):
            snd, buf, ssem, rsem, tgt = (
                (cw_send, cw_rs, cw_s_sems, cw_r_sems, right) if d
                else (ccw_send, ccw_rs, ccw_s_sems, ccw_r_sems, left)
            )
            return pltpu.make_async_remote_copy(
                src_ref=snd.at[h, :, seg_cols(s)],
                dst_ref=buf.at[h, :, seg_cols(s)],
                send_sem=ssem.at[h, s], recv_sem=rsem.at[h, s],
                device_id=(tgt,), device_id_type=pl.DeviceIdType.MESH,
            )

        def ag_rdma(h, d, s):
            buf, ssem, rsem, tgt = (
                (cw_ag, cw_ags_sems, cw_agr_sems, right) if d
                else (ccw_ag, ccw_ags_sems, ccw_agr_sems, left)
            )
            return pltpu.make_async_remote_copy(
                src_ref=buf.at[h, :, seg_cols(s)],
                dst_ref=buf.at[h + 1, :, seg_cols(s)],
                send_sem=ssem.at[h, s], recv_sem=rsem.at[h, s],
                device_id=(tgt,), device_id_type=pl.DeviceIdType.MESH,
            )

        rs = {(h, d, s): rs_rdma(h, d, s)
              for h in range(N_DEV - 1) for d in (True, False) for s in range(SEG)}
        ag = {(h, d, s): ag_rdma(h, d, s)
              for h in range(N_DEV - 1) for d in (True, False) for s in range(SEG)}

        for s in range(SEG):
            for d in (True, False):
                snd = cw_send if d else ccw_send
                snd[0, :, seg_cols(s)] = t_ref[
                    rows(my_pos), glob_cols(d, s)
                ].astype(jnp.bfloat16)
                rs[0, d, s].start()

        for h in range(N_DEV - 1):
            for s in range(SEG):
                for d in (True, False):
                    rs[h, d, s].wait_recv()
                    snd, buf = (cw_send, cw_rs) if d else (ccw_send, ccw_rs)
                    c_in = my_pos - h - 1 if d else my_pos + h + 1
                    partial = (
                        buf[h, :, seg_cols(s)].astype(jnp.float32)
                        + t_ref[rows(c_in), glob_cols(d, s)]
                    )
                    if h < N_DEV - 2:
                        snd[h + 1, :, seg_cols(s)] = partial.astype(jnp.bfloat16)
                        rs[h + 1, d, s].start()
                    else:
                        r = jnp.maximum(partial, 0.0)
                        fv = jnp.tanh(partial) * partial * partial + r * r * r
                        agb = cw_ag if d else ccw_ag
                        agb[0, :, seg_cols(s)] = fv.astype(jnp.bfloat16)
                        ag[0, d, s].start()
                        c_own = my_pos + 1 if d else my_pos - 1
                        out_ref[rows(c_own), glob_cols(d, s)] = fv

        for h in range(N_DEV - 1):
            for s in range(SEG):
                for d in (True, False):
                    ag[h, d, s].wait_recv()
                    if h < N_DEV - 2:
                        ag[h + 1, d, s].start()
                    agb = cw_ag if d else ccw_ag
                    c_recv = my_pos - h if d else my_pos + h
                    out_ref[rows(c_recv), glob_cols(d, s)] = agb[
                        h + 1, :, seg_cols(s)
                    ].astype(jnp.float32)

        for r_ in list(rs.values()) + list(ag.values()):
            r_.wait_send()

    half = (m_chunk, n_half)
    sem = pltpu.SemaphoreType.DMA((N_DEV - 1, SEG))
    return pl.pallas_call(
        body,
        out_shape=jax.ShapeDtypeStruct((m_per, n), jnp.float32),
        in_specs=[pl.BlockSpec(memory_space=pltpu.VMEM)],
        out_specs=pl.BlockSpec(memory_space=pltpu.VMEM),
        scratch_shapes=[
            pltpu.VMEM((N_DEV - 1,) + half, jnp.bfloat16),
            pltpu.VMEM((N_DEV - 1,) + half, jnp.bfloat16),
            pltpu.VMEM((N_DEV - 1,) + half, jnp.bfloat16),
            pltpu.VMEM((N_DEV - 1,) + half, jnp.bfloat16),
            pltpu.VMEM((N_DEV,) + half, jnp.bfloat16),
            pltpu.VMEM((N_DEV,) + half, jnp.bfloat16),
        ] + [sem for _ in range(8)],
        compiler_params=pltpu.CompilerParams(collective_id=0),
    )(t)
